# baseline (speedup 1.0000x reference)
import functools

import jax
import jax.numpy as jnp
from jax import lax
from jax.experimental import pallas as pl
from jax.experimental.pallas import tpu as pltpu
from jax.experimental.pallas import tpu_sc as plsc

N = 10000
E = 320000
EPAD = 327680
D_IN = 128
D_HID = 128
D_OUT = 40
D_OUT_PAD = 48

NC = 2
NS = 16
L = 16
NW = NC * NS
EW = EPAD // NW
CH = 80
NCHUNK = EW // CH
NP = 10240
RPT = NP // NS
TRASH = N

_mesh = plsc.VectorSubcoreMesh(core_axis_name="c", subcore_axis_name="s")


@functools.partial(
    pl.kernel,
    out_type=[
        jax.ShapeDtypeStruct((NC, NP, L), jnp.float32),
        jax.ShapeDtypeStruct((EPAD,), jnp.int32),
    ],
    mesh=_mesh,
    compiler_params=pltpu.CompilerParams(needs_layout_passes=False, use_tc_tiling_on_sc=False),
    scratch_types=[
        pltpu.VMEM((EW,), jnp.int32),
        pltpu.VMEM((EW,), jnp.int32),
        pltpu.VMEM((EW,), jnp.int32),
        pltpu.VMEM((CH,), jnp.int32),
        pltpu.VMEM((CH, L), jnp.float32),
        pltpu.VMEM_SHARED((NP, L), jnp.float32),
        pltpu.SemaphoreType.DMA,
    ],
)
def sc_degree(src_hbm, dst_hbm, out_hbm, ieff_hbm, sball, dball, iball, ibuf, onesbuf, acc, ssem):
    c = lax.axis_index("c")
    s = lax.axis_index("s")
    wid = c * NS + s

    pltpu.sync_copy(src_hbm.at[pl.ds(wid * EW, EW)], sball)
    pltpu.sync_copy(dst_hbm.at[pl.ds(wid * EW, EW)], dball)

    zero16 = jnp.zeros((L,), jnp.float32)
    for r in range(CH):
        onesbuf[r, pl.ds(0, L)] = zero16
    for k in range(RPT // CH):
        pltpu.sync_copy(onesbuf, acc.at[pl.ds(s * RPT + k * CH, CH)])

    e0 = jnp.where(lax.iota(jnp.int32, L) == 0, 1.0, 0.0)
    for r in range(CH):
        onesbuf[r, pl.ds(0, L)] = e0
    plsc.subcore_barrier()

    def fire(g, _):
        for j in range(CH // L):
            sv = sball[pl.ds(g * CH + j * L, L)]
            dv = dball[pl.ds(g * CH + j * L, L)]
            v = jnp.where(sv == dv, TRASH, dv)
            ibuf[pl.ds(j * L, L)] = v
            iball[pl.ds(g * CH + j * L, L)] = v
        pltpu.sync_copy(onesbuf, acc.at[ibuf], add=True)
        return 0

    lax.fori_loop(0, NCHUNK, fire, 0)

    pltpu.sync_copy(iball, ieff_hbm.at[pl.ds(wid * EW, EW)])
    plsc.subcore_barrier()

    pltpu.sync_copy(
        acc.at[pl.ds(s * RPT, RPT)], out_hbm.at[c, pl.ds(s * RPT, RPT)]
    )


NBUF = 4


def _make_sc_spmm(D):
    @functools.partial(
        pl.kernel,
        out_type=jax.ShapeDtypeStruct((NC, NP, D), jnp.float32),
        mesh=_mesh,
        compiler_params=pltpu.CompilerParams(needs_layout_passes=False, use_tc_tiling_on_sc=False),
        scratch_types=[
            [pltpu.VMEM((CH,), jnp.int32) for _ in range(NBUF)],
            [pltpu.VMEM((CH,), jnp.int32) for _ in range(NBUF)],
            [pltpu.VMEM((CH, D), jnp.float32) for _ in range(NBUF)],
            pltpu.VMEM_SHARED((NP, D), jnp.float32),
            [pltpu.SemaphoreType.DMA for _ in range(NBUF)],
            [pltpu.SemaphoreType.DMA for _ in range(NBUF)],
            [pltpu.SemaphoreType.DMA for _ in range(NBUF)],
        ],
    )
    def sc_spmm(h_hbm, src_hbm, ieff_hbm, out_hbm,
                sbufs, ibufs, rowbufs, acc, isems, gsems, ssems):
        c = lax.axis_index("c")
        s = lax.axis_index("s")
        wid = c * NS + s

        def issue_idx(g, slot):
            pltpu.async_copy(
                src_hbm.at[pl.ds(wid * EW + g * CH, CH)], sbufs[slot], isems[slot])
            pltpu.async_copy(
                ieff_hbm.at[pl.ds(wid * EW + g * CH, CH)], ibufs[slot], isems[slot])

        def wait_idx(g, slot):
            pltpu.make_async_copy(
                src_hbm.at[pl.ds(wid * EW + g * CH, CH)], sbufs[slot], isems[slot]).wait()
            pltpu.make_async_copy(
                ieff_hbm.at[pl.ds(wid * EW + g * CH, CH)], ibufs[slot], isems[slot]).wait()

        def issue_gather(slot):
            pltpu.async_copy(h_hbm.at[sbufs[slot]], rowbufs[slot], gsems[slot])

        def wait_gather(slot):
            pltpu.make_async_copy(h_hbm.at[sbufs[slot]], rowbufs[slot], gsems[slot]).wait()

        def issue_scatter(slot):
            pltpu.async_copy(rowbufs[slot], acc.at[ibufs[slot]], ssems[slot], add=True)

        def wait_scatter(slot):
            pltpu.make_async_copy(rowbufs[slot], acc.at[ibufs[slot]], ssems[slot]).wait()

        zero16 = jnp.zeros((L,), jnp.float32)
        zbuf = rowbufs[0]
        for r in range(CH):
            for cc in range(D // L):
                zbuf[r, pl.ds(cc * L, L)] = zero16
        for k in range(RPT // CH):
            pltpu.sync_copy(zbuf, acc.at[pl.ds(s * RPT + k * CH, CH)])

        for b in range(NBUF - 1):
            issue_idx(b, b)
        for b in range(2):
            wait_idx(b, b)
            issue_gather(b)
        plsc.subcore_barrier()

        def outer(i5, _):
            for b in range(NBUF):
                g = i5 * NBUF + b
                bn = (b + NBUF - 1) % NBUF
                b2 = (b + 2) % NBUF

                @pl.when(g >= 1)
                def _():
                    wait_scatter(bn)

                @pl.when(g + NBUF - 1 < NCHUNK)
                def _():
                    issue_idx(g + NBUF - 1, bn)

                @pl.when(g + 2 < NCHUNK)
                def _():
                    wait_idx(g + 2, b2)
                    issue_gather(b2)

                wait_gather(b)
                issue_scatter(b)
            return 0

        lax.fori_loop(0, NCHUNK // NBUF, outer, 0)
        wait_scatter((NCHUNK - 1) % NBUF)
        plsc.subcore_barrier()

        pltpu.sync_copy(
            acc.at[pl.ds(s * RPT, RPT)], out_hbm.at[c, pl.ds(s * RPT, RPT)]
        )

    return sc_spmm


_sc_spmm_hid = _make_sc_spmm(D_HID)
_sc_spmm_out = _make_sc_spmm(D_OUT_PAD)


BR = 1024
GRID = NP // BR


def _tc1_body(x_ref, w1_ref, dp_ref, h1p_ref, dinv_ref):
    deg = dp_ref[0, :, 0] + dp_ref[1, :, 0] + 1.0
    dinv = lax.rsqrt(deg)
    h = jnp.dot(x_ref[...], w1_ref[...], precision=lax.Precision.HIGHEST)
    h1p_ref[...] = h * dinv[:, None]
    dinv_ref[...] = dinv[:, None]


def _tc1(x, W1, deg_parts):
    return pl.pallas_call(
        _tc1_body,
        grid=(GRID,),
        in_specs=[
            pl.BlockSpec((BR, D_IN), lambda i: (i, 0)),
            pl.BlockSpec((D_IN, D_HID), lambda i: (0, 0)),
            pl.BlockSpec((NC, BR, L), lambda i: (0, i, 0)),
        ],
        out_specs=[
            pl.BlockSpec((BR, D_HID), lambda i: (i, 0)),
            pl.BlockSpec((BR, 1), lambda i: (i, 0)),
        ],
        out_shape=[
            jax.ShapeDtypeStruct((NP, D_HID), jnp.float32),
            jax.ShapeDtypeStruct((NP, 1), jnp.float32),
        ],
    )(x, W1, deg_parts)


def _tc2_body(acc_ref, h1p_ref, dinv_ref, b1_ref, w2_ref, h2p_ref):
    dinv = dinv_ref[...]
    ssum = acc_ref[0] + acc_ref[1] + h1p_ref[...]
    o1 = jnp.maximum(ssum * dinv + b1_ref[...], 0.0)
    h2 = jnp.dot(o1, w2_ref[...], precision=lax.Precision.HIGHEST)
    h2p_ref[...] = h2 * dinv


def _tc2(acc1, h1p, dinv, b1r, W2pad):
    return pl.pallas_call(
        _tc2_body,
        grid=(GRID,),
        in_specs=[
            pl.BlockSpec((NC, BR, D_HID), lambda i: (0, i, 0)),
            pl.BlockSpec((BR, D_HID), lambda i: (i, 0)),
            pl.BlockSpec((BR, 1), lambda i: (i, 0)),
            pl.BlockSpec((1, D_HID), lambda i: (0, 0)),
            pl.BlockSpec((D_HID, D_OUT_PAD), lambda i: (0, 0)),
        ],
        out_specs=pl.BlockSpec((BR, D_OUT_PAD), lambda i: (i, 0)),
        out_shape=jax.ShapeDtypeStruct((NP, D_OUT_PAD), jnp.float32),
    )(acc1, h1p, dinv, b1r, W2pad)


def _tc3_body(acc_ref, h2p_ref, dinv_ref, b2_ref, out_ref):
    y = (acc_ref[0] + acc_ref[1] + h2p_ref[...]) * dinv_ref[...] + b2_ref[...]
    col = lax.broadcasted_iota(jnp.int32, (BR, D_OUT_PAD), 1)
    valid = col < D_OUT
    ym = jnp.where(valid, y, -jnp.inf)
    mx = jnp.max(ym, axis=1, keepdims=True)
    ex = jnp.where(valid, jnp.exp(ym - mx), 0.0)
    lse = jnp.log(jnp.sum(ex, axis=1, keepdims=True))
    res = y - mx - lse
    out_ref[...] = res[:, :D_OUT]


def _tc3(acc2, h2p, dinv, b2r):
    return pl.pallas_call(
        _tc3_body,
        grid=(GRID,),
        in_specs=[
            pl.BlockSpec((NC, BR, D_OUT_PAD), lambda i: (0, i, 0)),
            pl.BlockSpec((BR, D_OUT_PAD), lambda i: (i, 0)),
            pl.BlockSpec((BR, 1), lambda i: (i, 0)),
            pl.BlockSpec((1, D_OUT_PAD), lambda i: (0, 0)),
        ],
        out_specs=pl.BlockSpec((BR, D_OUT), lambda i: (i, 0)),
        out_shape=jax.ShapeDtypeStruct((NP, D_OUT), jnp.float32),
    )(acc2, h2p, dinv, b2r)


def kernel(features, edge_index, W1, b1, W2, b2):
    src = jnp.pad(edge_index[0], (0, EPAD - E))
    dst = jnp.pad(edge_index[1], (0, EPAD - E))

    W2pad = jnp.pad(W2, ((0, 0), (0, D_OUT_PAD - D_OUT)))
    b1r = b1.reshape(1, D_HID)
    b2r = jnp.pad(b2, (0, D_OUT_PAD - D_OUT)).reshape(1, D_OUT_PAD)

    x_pad = jnp.pad(features, ((0, NP - N), (0, 0)))

    deg_parts, ieff = sc_degree(src, dst)
    h1p, dinv = _tc1(x_pad, W1, deg_parts)
    acc1 = _sc_spmm_hid(h1p, src, ieff)
    h2p = _tc2(acc1, h1p, dinv, b1r, W2pad)
    acc2 = _sc_spmm_out(h2p, src, ieff)
    return _tc3(acc2, h2p, dinv, b2r)[:N]


# speedup vs baseline: 1.0310x; 1.0310x over previous
import functools

import jax
import jax.numpy as jnp
from jax import lax
from jax.experimental import pallas as pl
from jax.experimental.pallas import tpu as pltpu
from jax.experimental.pallas import tpu_sc as plsc

N = 10000
E = 320000
EPAD = 327680
D_IN = 128
D_HID = 128
D_OUT = 40
D_OUT_PAD = 48

NC = 2
NS = 16
L = 16
NW = NC * NS
EW = EPAD // NW
CH = 80
NCHUNK = EW // CH
NP = 10240
RPT = NP // NS
TRASH = N

_mesh = plsc.VectorSubcoreMesh(core_axis_name="c", subcore_axis_name="s")


@functools.partial(
    pl.kernel,
    out_type=[
        jax.ShapeDtypeStruct((NC, NP, L), jnp.float32),
        jax.ShapeDtypeStruct((EPAD,), jnp.int32),
    ],
    mesh=_mesh,
    compiler_params=pltpu.CompilerParams(needs_layout_passes=False, use_tc_tiling_on_sc=False),
    scratch_types=[
        pltpu.VMEM((EW,), jnp.int32),
        pltpu.VMEM((EW,), jnp.int32),
        pltpu.VMEM((EW,), jnp.int32),
        pltpu.VMEM((CH,), jnp.int32),
        pltpu.VMEM((CH, L), jnp.float32),
        pltpu.VMEM_SHARED((NP, L), jnp.float32),
        pltpu.SemaphoreType.DMA,
    ],
)
def sc_degree(src_hbm, dst_hbm, out_hbm, ieff_hbm, sball, dball, iball, ibuf, onesbuf, acc, ssem):
    c = lax.axis_index("c")
    s = lax.axis_index("s")
    wid = c * NS + s

    pltpu.sync_copy(src_hbm.at[pl.ds(wid * EW, EW)], sball)
    pltpu.sync_copy(dst_hbm.at[pl.ds(wid * EW, EW)], dball)

    zero16 = jnp.zeros((L,), jnp.float32)
    for r in range(CH):
        onesbuf[r, pl.ds(0, L)] = zero16
    for k in range(RPT // CH):
        pltpu.sync_copy(onesbuf, acc.at[pl.ds(s * RPT + k * CH, CH)])

    e0 = jnp.where(lax.iota(jnp.int32, L) == 0, 1.0, 0.0)
    for r in range(CH):
        onesbuf[r, pl.ds(0, L)] = e0
    plsc.subcore_barrier()

    def fire(g, _):
        for j in range(CH // L):
            sv = sball[pl.ds(g * CH + j * L, L)]
            dv = dball[pl.ds(g * CH + j * L, L)]
            v = jnp.where(sv == dv, TRASH, dv)
            ibuf[pl.ds(j * L, L)] = v
            iball[pl.ds(g * CH + j * L, L)] = v
        pltpu.sync_copy(onesbuf, acc.at[ibuf], add=True)
        return 0

    lax.fori_loop(0, NCHUNK, fire, 0)

    pltpu.sync_copy(iball, ieff_hbm.at[pl.ds(wid * EW, EW)])
    plsc.subcore_barrier()

    pltpu.sync_copy(
        acc.at[pl.ds(s * RPT, RPT)], out_hbm.at[c, pl.ds(s * RPT, RPT)]
    )


EW0 = 15360
EW1 = 5120
NBUF = 4


def _make_sc_spmm(D):
    @functools.partial(
        pl.kernel,
        out_type=jax.ShapeDtypeStruct((NC, NP, D), jnp.float32),
        mesh=_mesh,
        compiler_params=pltpu.CompilerParams(needs_layout_passes=False, use_tc_tiling_on_sc=False),
        scratch_types=[
            [pltpu.VMEM((CH,), jnp.int32) for _ in range(NBUF)],
            [pltpu.VMEM((CH,), jnp.int32) for _ in range(NBUF)],
            [pltpu.VMEM((CH, D), jnp.float32) for _ in range(NBUF)],
            pltpu.VMEM_SHARED((NP, D), jnp.float32),
            [pltpu.SemaphoreType.DMA for _ in range(NBUF)],
            [pltpu.SemaphoreType.DMA for _ in range(NBUF)],
            [pltpu.SemaphoreType.DMA for _ in range(NBUF)],
        ],
    )
    def sc_spmm(h_hbm, src_hbm, ieff_hbm, out_hbm,
                sbufs, ibufs, rowbufs, acc, isems, gsems, ssems):
        c = lax.axis_index("c")
        s = lax.axis_index("s")
        base = jnp.where(c == 0, s * EW0, NS * EW0 + s * EW1)
        nchunk = jnp.where(c == 0, EW0 // CH, EW1 // CH)

        def issue_idx(g, slot):
            pltpu.async_copy(
                src_hbm.at[pl.ds(base + g * CH, CH)], sbufs[slot], isems[slot])
            pltpu.async_copy(
                ieff_hbm.at[pl.ds(base + g * CH, CH)], ibufs[slot], isems[slot])

        def wait_idx(g, slot):
            pltpu.make_async_copy(
                src_hbm.at[pl.ds(base + g * CH, CH)], sbufs[slot], isems[slot]).wait()
            pltpu.make_async_copy(
                ieff_hbm.at[pl.ds(base + g * CH, CH)], ibufs[slot], isems[slot]).wait()

        def issue_gather(slot):
            pltpu.async_copy(h_hbm.at[sbufs[slot]], rowbufs[slot], gsems[slot])

        def wait_gather(slot):
            pltpu.make_async_copy(h_hbm.at[sbufs[slot]], rowbufs[slot], gsems[slot]).wait()

        def issue_scatter(slot):
            pltpu.async_copy(rowbufs[slot], acc.at[ibufs[slot]], ssems[slot], add=True)

        def wait_scatter(slot):
            pltpu.make_async_copy(rowbufs[slot], acc.at[ibufs[slot]], ssems[slot]).wait()

        zero16 = jnp.zeros((L,), jnp.float32)
        zbuf = rowbufs[0]
        for r in range(CH):
            for cc in range(D // L):
                zbuf[r, pl.ds(cc * L, L)] = zero16
        for k in range(RPT // CH):
            pltpu.sync_copy(zbuf, acc.at[pl.ds(s * RPT + k * CH, CH)])

        for b in range(NBUF - 1):
            issue_idx(b, b)
        for b in range(2):
            wait_idx(b, b)
            issue_gather(b)
        plsc.subcore_barrier()

        def outer(i5, _):
            for b in range(NBUF):
                g = i5 * NBUF + b
                bn = (b + NBUF - 1) % NBUF
                b2 = (b + 2) % NBUF

                @pl.when(g >= 1)
                def _():
                    wait_scatter(bn)

                @pl.when(g + NBUF - 1 < nchunk)
                def _():
                    issue_idx(g + NBUF - 1, bn)

                @pl.when(g + 2 < nchunk)
                def _():
                    wait_idx(g + 2, b2)
                    issue_gather(b2)

                wait_gather(b)
                issue_scatter(b)
            return 0

        lax.fori_loop(0, nchunk // NBUF, outer, 0)
        wait_scatter(NBUF - 1)
        plsc.subcore_barrier()

        pltpu.sync_copy(
            acc.at[pl.ds(s * RPT, RPT)], out_hbm.at[c, pl.ds(s * RPT, RPT)]
        )

    return sc_spmm


_sc_spmm_hid = _make_sc_spmm(D_HID)
_sc_spmm_out = _make_sc_spmm(D_OUT_PAD)


BR = 1024
GRID = NP // BR


def _tc1_body(x_ref, w1_ref, dp_ref, h1p_ref, dinv_ref):
    deg = dp_ref[0, :, 0] + dp_ref[1, :, 0] + 1.0
    dinv = lax.rsqrt(deg)
    h = jnp.dot(x_ref[...], w1_ref[...], precision=lax.Precision.HIGHEST)
    h1p_ref[...] = h * dinv[:, None]
    dinv_ref[...] = dinv[:, None]


def _tc1(x, W1, deg_parts):
    return pl.pallas_call(
        _tc1_body,
        grid=(GRID,),
        in_specs=[
            pl.BlockSpec((BR, D_IN), lambda i: (i, 0)),
            pl.BlockSpec((D_IN, D_HID), lambda i: (0, 0)),
            pl.BlockSpec((NC, BR, L), lambda i: (0, i, 0)),
        ],
        out_specs=[
            pl.BlockSpec((BR, D_HID), lambda i: (i, 0)),
            pl.BlockSpec((BR, 1), lambda i: (i, 0)),
        ],
        out_shape=[
            jax.ShapeDtypeStruct((NP, D_HID), jnp.float32),
            jax.ShapeDtypeStruct((NP, 1), jnp.float32),
        ],
    )(x, W1, deg_parts)


def _tc2_body(acc_ref, h1p_ref, dinv_ref, b1_ref, w2_ref, h2p_ref):
    dinv = dinv_ref[...]
    ssum = acc_ref[0] + acc_ref[1] + h1p_ref[...]
    o1 = jnp.maximum(ssum * dinv + b1_ref[...], 0.0)
    h2 = jnp.dot(o1, w2_ref[...], precision=lax.Precision.HIGHEST)
    h2p_ref[...] = h2 * dinv


def _tc2(acc1, h1p, dinv, b1r, W2pad):
    return pl.pallas_call(
        _tc2_body,
        grid=(GRID,),
        in_specs=[
            pl.BlockSpec((NC, BR, D_HID), lambda i: (0, i, 0)),
            pl.BlockSpec((BR, D_HID), lambda i: (i, 0)),
            pl.BlockSpec((BR, 1), lambda i: (i, 0)),
            pl.BlockSpec((1, D_HID), lambda i: (0, 0)),
            pl.BlockSpec((D_HID, D_OUT_PAD), lambda i: (0, 0)),
        ],
        out_specs=pl.BlockSpec((BR, D_OUT_PAD), lambda i: (i, 0)),
        out_shape=jax.ShapeDtypeStruct((NP, D_OUT_PAD), jnp.float32),
    )(acc1, h1p, dinv, b1r, W2pad)


def _tc3_body(acc_ref, h2p_ref, dinv_ref, b2_ref, out_ref):
    y = (acc_ref[0] + acc_ref[1] + h2p_ref[...]) * dinv_ref[...] + b2_ref[...]
    col = lax.broadcasted_iota(jnp.int32, (BR, D_OUT_PAD), 1)
    valid = col < D_OUT
    ym = jnp.where(valid, y, -jnp.inf)
    mx = jnp.max(ym, axis=1, keepdims=True)
    ex = jnp.where(valid, jnp.exp(ym - mx), 0.0)
    lse = jnp.log(jnp.sum(ex, axis=1, keepdims=True))
    res = y - mx - lse
    out_ref[...] = res[:, :D_OUT]


def _tc3(acc2, h2p, dinv, b2r):
    return pl.pallas_call(
        _tc3_body,
        grid=(GRID,),
        in_specs=[
            pl.BlockSpec((NC, BR, D_OUT_PAD), lambda i: (0, i, 0)),
            pl.BlockSpec((BR, D_OUT_PAD), lambda i: (i, 0)),
            pl.BlockSpec((BR, 1), lambda i: (i, 0)),
            pl.BlockSpec((1, D_OUT_PAD), lambda i: (0, 0)),
        ],
        out_specs=pl.BlockSpec((BR, D_OUT), lambda i: (i, 0)),
        out_shape=jax.ShapeDtypeStruct((NP, D_OUT), jnp.float32),
    )(acc2, h2p, dinv, b2r)


def kernel(features, edge_index, W1, b1, W2, b2):
    src = jnp.pad(edge_index[0], (0, EPAD - E))
    dst = jnp.pad(edge_index[1], (0, EPAD - E))

    W2pad = jnp.pad(W2, ((0, 0), (0, D_OUT_PAD - D_OUT)))
    b1r = b1.reshape(1, D_HID)
    b2r = jnp.pad(b2, (0, D_OUT_PAD - D_OUT)).reshape(1, D_OUT_PAD)

    x_pad = jnp.pad(features, ((0, NP - N), (0, 0)))

    deg_parts, ieff = sc_degree(src, dst)
    h1p, dinv = _tc1(x_pad, W1, deg_parts)
    acc1 = _sc_spmm_hid(h1p, src, ieff)
    h2p = _tc2(acc1, h1p, dinv, b1r, W2pad)
    acc2 = _sc_spmm_out(h2p, src, ieff)
    return _tc3(acc2, h2p, dinv, b2r)[:N]
